# baseline (speedup 1.0000x reference)
import functools

import jax
import jax.numpy as jnp
from jax import lax
from jax.experimental import pallas as pl
from jax.experimental.pallas import tpu as pltpu
from jax.experimental.pallas import tpu_sc as plsc

_BETA = 0.25
_TM = 1024


def _dist_argmin_body(x_ref, cbT_ref, idx_ref, lsum_ref, c2_ref, acc_ref,
                      *, num_codes):
    i = pl.program_id(0)
    nsteps = pl.num_programs(0)

    @pl.when(i == 0)
    def _init():
        cb = cbT_ref[...]
        c2_ref[...] = jnp.sum(cb * cb, axis=0, keepdims=True)
        acc_ref[0] = 0.0

    xb = x_ref[...]
    ab = jnp.dot(xb, cbT_ref[...], preferred_element_type=jnp.float32)
    x2 = jnp.sum(xb * xb, axis=1, keepdims=True)
    d = (x2 - 2.0 * ab) + c2_ref[...]
    m = jnp.min(d, axis=1, keepdims=True)
    ids = lax.broadcasted_iota(jnp.int32, d.shape, 1)
    idx = jnp.min(jnp.where(d == m, ids, num_codes), axis=1)
    idx_ref[...] = idx
    acc_ref[0] += jnp.sum(m[:, 0])

    @pl.when(i == nsteps - 1)
    def _fin():
        lsum_ref[0] = acc_ref[0]


def _dist_argmin(flat, cbT):
    T, D = flat.shape
    num_codes = cbT.shape[1]
    body = functools.partial(_dist_argmin_body, num_codes=num_codes)
    return pl.pallas_call(
        body,
        grid=(T // _TM,),
        in_specs=[
            pl.BlockSpec((_TM, D), lambda i: (i, 0)),
            pl.BlockSpec((D, num_codes), lambda i: (0, 0)),
        ],
        out_specs=[
            pl.BlockSpec((_TM,), lambda i: (i,)),
            pl.BlockSpec(memory_space=pltpu.SMEM),
        ],
        out_shape=[
            jax.ShapeDtypeStruct((T,), jnp.int32),
            jax.ShapeDtypeStruct((1,), jnp.float32),
        ],
        scratch_shapes=[
            pltpu.VMEM((1, num_codes), jnp.float32),
            pltpu.SMEM((1,), jnp.float32),
        ],
    )(flat, cbT)


_CH = 128


def _make_gather(T, D):
    info = plsc.get_sparse_core_info()
    nw = info.num_cores * info.num_subcores
    bpw = T // nw
    nchunks = bpw // _CH
    mesh = plsc.VectorSubcoreMesh(core_axis_name="c", subcore_axis_name="s")

    @functools.partial(
        pl.kernel, mesh=mesh,
        out_type=jax.ShapeDtypeStruct((T, D), jnp.float32),
        scratch_types=[
            pltpu.VMEM((_CH,), jnp.int32),
            pltpu.VMEM((_CH, D), jnp.float32),
            pltpu.SemaphoreType.DMA,
        ],
    )
    def _gather(cb_hbm, idx_hbm, out_hbm, idx_v, rows_v, sem):
        wid = lax.axis_index("s") * info.num_cores + lax.axis_index("c")
        base = wid * bpw
        for t in range(nchunks):
            pltpu.sync_copy(idx_hbm.at[pl.ds(base + t * _CH, _CH)], idx_v)
            pltpu.async_copy(cb_hbm.at[idx_v], rows_v, sem).wait()
            pltpu.sync_copy(rows_v, out_hbm.at[pl.ds(base + t * _CH, _CH)])

    return _gather


def kernel(x, codebook):
    B, N, D = x.shape
    T = B * N
    flat = x.reshape(T, D)
    cbT = codebook.T
    half = T // 2
    gather = _make_gather(half, D)
    idx0, s0 = _dist_argmin(flat[:half], cbT)
    zq0 = gather(codebook, idx0)
    idx1, s1 = _dist_argmin(flat[half:], cbT)
    zq1 = gather(codebook, idx1)
    mean = (s0[0] + s1[0]) / float(T * D)
    loss = mean + _BETA * mean
    zq = jnp.concatenate([zq0, zq1], axis=0).reshape(B, N, D)
    idx_flat = jnp.concatenate([idx0, idx1], axis=0)
    return zq, loss, idx_flat.reshape(B, N)


# speedup vs baseline: 1.1195x; 1.1195x over previous
import functools

import jax
import jax.numpy as jnp
from jax import lax
from jax.experimental import pallas as pl
from jax.experimental.pallas import tpu as pltpu
from jax.experimental.pallas import tpu_sc as plsc

_BETA = 0.25
_TM = 1024


def _dist_argmin_body(x_ref, cbT_ref, idx_ref, loss_ref, c2_ref, acc_ref,
                      *, num_codes, total_elems):
    i = pl.program_id(0)
    nsteps = pl.num_programs(0)

    @pl.when(i == 0)
    def _init():
        cb = cbT_ref[...]
        c2_ref[...] = jnp.sum(cb * cb, axis=0, keepdims=True)
        acc_ref[0] = 0.0

    xb = x_ref[...]
    ab = jnp.dot(xb, cbT_ref[...], preferred_element_type=jnp.float32)
    x2 = jnp.sum(xb * xb, axis=1, keepdims=True)
    d = (x2 - 2.0 * ab) + c2_ref[...]
    m = jnp.min(d, axis=1, keepdims=True)
    ids = lax.broadcasted_iota(jnp.int32, d.shape, 1)
    idx = jnp.min(jnp.where(d == m, ids, num_codes), axis=1)
    idx_ref[...] = idx
    acc_ref[0] += jnp.sum(m[:, 0])

    @pl.when(i == nsteps - 1)
    def _fin():
        mean = acc_ref[0] / total_elems
        loss_ref[0] = mean + _BETA * mean


def _dist_argmin(flat, cbT):
    T, D = flat.shape
    num_codes = cbT.shape[1]
    body = functools.partial(_dist_argmin_body, num_codes=num_codes,
                             total_elems=float(T * D))
    return pl.pallas_call(
        body,
        grid=(T // _TM,),
        in_specs=[
            pl.BlockSpec((_TM, D), lambda i: (i, 0)),
            pl.BlockSpec((D, num_codes), lambda i: (0, 0)),
        ],
        out_specs=[
            pl.BlockSpec((_TM,), lambda i: (i,)),
            pl.BlockSpec(memory_space=pltpu.SMEM),
        ],
        out_shape=[
            jax.ShapeDtypeStruct((T,), jnp.int32),
            jax.ShapeDtypeStruct((1,), jnp.float32),
        ],
        scratch_shapes=[
            pltpu.VMEM((1, num_codes), jnp.float32),
            pltpu.SMEM((1,), jnp.float32),
        ],
    )(flat, cbT)


_CH = 128


def _make_gather(T, D):
    info = plsc.get_sparse_core_info()
    nw = info.num_cores * info.num_subcores
    bpw = T // nw
    nchunks = bpw // _CH
    mesh = plsc.VectorSubcoreMesh(core_axis_name="c", subcore_axis_name="s")

    @functools.partial(
        pl.kernel, mesh=mesh,
        out_type=jax.ShapeDtypeStruct((T, D), jnp.float32),
        scratch_types=[
            pltpu.VMEM((_CH,), jnp.int32),
            pltpu.VMEM((_CH, D), jnp.float32),
            pltpu.SemaphoreType.DMA,
        ],
    )
    def _gather(cb_hbm, idx_hbm, out_hbm, idx_v, rows_v, sem):
        wid = lax.axis_index("s") * info.num_cores + lax.axis_index("c")
        base = wid * bpw
        for t in range(nchunks):
            pltpu.sync_copy(idx_hbm.at[pl.ds(base + t * _CH, _CH)], idx_v)
            pltpu.async_copy(cb_hbm.at[idx_v], rows_v, sem).wait()
            pltpu.sync_copy(rows_v, out_hbm.at[pl.ds(base + t * _CH, _CH)])

    return _gather


def kernel(x, codebook):
    B, N, D = x.shape
    flat = x.reshape(B * N, D)
    cbT = codebook.T
    idx_flat, loss = _dist_argmin(flat, cbT)
    zq = _make_gather(B * N, D)(codebook, idx_flat)
    return zq.reshape(B, N, D), loss[0], idx_flat.reshape(B, N)
